# baseline (speedup 1.0000x reference)
import functools

import jax
import jax.numpy as jnp
from jax import lax
from jax.experimental import pallas as pl
from jax.experimental.pallas import tpu as pltpu
from jax.experimental.pallas import tpu_sc as plsc

N_FACTORS = 32
BATCH = 16384
LANES = 16
NUM_WORKERS = 32
B_PER_W = BATCH // NUM_WORKERS
GROUPS = B_PER_W // LANES


@functools.partial(
    pl.kernel,
    mesh=plsc.VectorSubcoreMesh(core_axis_name="c", subcore_axis_name="s"),
    out_type=jax.ShapeDtypeStruct((BATCH,), jnp.float32),
    scratch_types=[
        pltpu.VMEM((B_PER_W,), jnp.int32),
        pltpu.VMEM((B_PER_W,), jnp.int32),
        pltpu.VMEM((B_PER_W, N_FACTORS), jnp.float32),
        pltpu.VMEM((B_PER_W, N_FACTORS), jnp.float32),
        pltpu.VMEM((B_PER_W,), jnp.float32),
        pltpu.VMEM((B_PER_W,), jnp.float32),
        pltpu.VMEM((B_PER_W,), jnp.float32),
        pltpu.SemaphoreType.DMA,
    ],
    compiler_params=pltpu.CompilerParams(
        needs_layout_passes=False, use_tc_tiling_on_sc=False),
)
def _mf_sc_kernel(user_hbm, item_hbm, uf_hbm, if_hbm, ub_hbm, ib_hbm,
                  out_hbm, uidx_v, iidx_v, urows_v, irows_v, ub_v, ib_v,
                  out_v, sem):
    info = plsc.get_sparse_core_info()
    wid = lax.axis_index("s") * info.num_cores + lax.axis_index("c")
    base = wid * B_PER_W

    pltpu.sync_copy(user_hbm.at[pl.ds(base, B_PER_W)], uidx_v)
    pltpu.sync_copy(item_hbm.at[pl.ds(base, B_PER_W)], iidx_v)

    cps = [
        pltpu.async_copy(uf_hbm.at[uidx_v], urows_v, sem),
        pltpu.async_copy(if_hbm.at[iidx_v], irows_v, sem),
        pltpu.async_copy(ub_hbm.at[uidx_v], ub_v, sem),
        pltpu.async_copy(ib_hbm.at[iidx_v], ib_v, sem),
    ]
    for cp in cps:
        cp.wait()

    def group_body(g, carry):
        rbase = g * LANES
        rows = rbase + lax.iota(jnp.int32, LANES)
        acc = ub_v[pl.ds(rbase, LANES)] + ib_v[pl.ds(rbase, LANES)]
        for f in range(N_FACTORS):
            fcol = jnp.full((LANES,), f, jnp.int32)
            u = plsc.load_gather(urows_v, [rows, fcol])
            it = plsc.load_gather(irows_v, [rows, fcol])
            acc = acc + u * it
        out_v[pl.ds(rbase, LANES)] = acc
        return carry

    lax.fori_loop(0, GROUPS, group_body, 0)
    pltpu.sync_copy(out_v, out_hbm.at[pl.ds(base, B_PER_W)])


def kernel(user, item, user_factors, item_factors, user_biases, item_biases):
    ub = user_biases.reshape(-1)
    ib = item_biases.reshape(-1)
    return _mf_sc_kernel(user, item, user_factors, item_factors, ub, ib)
